# Initial kernel scaffold; baseline (speedup 1.0000x reference)
import jax
import jax.numpy as jnp
from jax.experimental import pallas as pl


def kernel(x_chemical, x_gene, edge_index_cg, edge_index_gc, Wsrc, bsrc, Wdst, bdst, attn, Wout, bout):
    raise NotImplementedError("write your pallas kernel here")


# speedup vs baseline: 6.6738x; 6.6738x over previous
import functools

import jax
import jax.numpy as jnp
from jax import lax
from jax.experimental import pallas as pl
from jax.experimental.pallas import tpu as pltpu
from jax.experimental.pallas import tpu_sc as plsc

N = 10000
E = 320000
C = 128
H = 4
DH = 32
L = 2

NSUB = 16
EW = E // NSUB
K = 80
NCH = EW // K
RW = 624
RREM = N - RW * NSUB
RB = 1000

_f32 = jnp.float32


def _proj_body(xc, xg, Wcg, bcg, Wgc, bgc, Mc, Mg, b16, hs_cg, hs_gc, stab):
    xcb = xc[...]
    xgb = xg[...]
    hs_cg[...] = jnp.dot(xcb, Wcg[...], preferred_element_type=_f32) + bcg[...]
    hs_gc[...] = jnp.dot(xgb, Wgc[...], preferred_element_type=_f32) + bgc[...]
    stab[...] = (jnp.dot(xcb, Mc[...], preferred_element_type=_f32)
                 + jnp.dot(xgb, Mg[...], preferred_element_type=_f32)
                 + b16[...])


def _proj_call(xc, xg, Wcg, bcg, Wgc, bgc, Mc, Mg, b16):
    row = lambda i: (i, 0)
    full = lambda i: (0, 0)
    return pl.pallas_call(
        _proj_body,
        grid=(N // RB,),
        in_specs=[
            pl.BlockSpec((RB, C), row), pl.BlockSpec((RB, C), row),
            pl.BlockSpec((C, C), full), pl.BlockSpec((1, C), full),
            pl.BlockSpec((C, C), full), pl.BlockSpec((1, C), full),
            pl.BlockSpec((C, 16), full), pl.BlockSpec((C, 16), full),
            pl.BlockSpec((1, 16), full),
        ],
        out_specs=[pl.BlockSpec((RB, C), row), pl.BlockSpec((RB, C), row),
                   pl.BlockSpec((RB, 16), row)],
        out_shape=[jax.ShapeDtypeStruct((N, C), _f32),
                   jax.ShapeDtypeStruct((N, C), _f32),
                   jax.ShapeDtypeStruct((N, 16), _f32)],
    )(xc, xg, Wcg, bcg, Wgc, bgc, Mc, Mg, b16)


def _out_body(aggg, deng, aggc, denc, Wg, bg, Wc, bc, xg, xc, Ex, yg, yc):
    ex = Ex[...]
    sg = jnp.dot(1.0 / (deng[...] + 1e-16), ex, preferred_element_type=_f32)
    ag = aggg[...] * sg
    yg[...] = jnp.maximum(
        jnp.dot(ag, Wg[...], preferred_element_type=_f32) + bg[...], 0.0) + xg[...]
    sc = jnp.dot(1.0 / (denc[...] + 1e-16), ex, preferred_element_type=_f32)
    ac = aggc[...] * sc
    yc[...] = jnp.maximum(
        jnp.dot(ac, Wc[...], preferred_element_type=_f32) + bc[...], 0.0) + xc[...]


def _out_call(aggg, deng, aggc, denc, Wg, bg, Wc, bc, xg, xc, Ex):
    row = lambda i: (i, 0)
    full = lambda i: (0, 0)
    return pl.pallas_call(
        _out_body,
        grid=(N // RB,),
        in_specs=[
            pl.BlockSpec((RB, C), row), pl.BlockSpec((RB, 16), row),
            pl.BlockSpec((RB, C), row), pl.BlockSpec((RB, 16), row),
            pl.BlockSpec((C, C), full), pl.BlockSpec((1, C), full),
            pl.BlockSpec((C, C), full), pl.BlockSpec((1, C), full),
            pl.BlockSpec((RB, C), row), pl.BlockSpec((RB, C), row),
            pl.BlockSpec((16, C), full),
        ],
        out_specs=[pl.BlockSpec((RB, C), row), pl.BlockSpec((RB, C), row)],
        out_shape=[jax.ShapeDtypeStruct((N, C), _f32),
                   jax.ShapeDtypeStruct((N, C), _f32)],
    )(aggg, deng, aggc, denc, Wg, bg, Wc, bc, xg, xc, Ex)


def _sc_edge(hs_cg, hs_gc, stab, src_cg, dst_cg, src_gc, dst_gc):
    mesh = plsc.VectorSubcoreMesh(core_axis_name="c", subcore_axis_name="s")
    out_type = [jax.ShapeDtypeStruct((N, C), _f32),
                jax.ShapeDtypeStruct((N, 16), _f32),
                jax.ShapeDtypeStruct((N, C), _f32),
                jax.ShapeDtypeStruct((N, 16), _f32)]
    scratch = [
        pltpu.VMEM((K, C), _f32),
        pltpu.VMEM((K, 16), _f32),
        pltpu.VMEM((K, 16), _f32),
        pltpu.VMEM((4 * K,), _f32),
        pltpu.VMEM((K, 16), _f32),
        pltpu.VMEM((K,), jnp.int32),
        pltpu.VMEM((K,), jnp.int32),
        pltpu.VMEM_SHARED((N, C), _f32),
        pltpu.VMEM_SHARED((N, 16), _f32),
        pltpu.SemaphoreType.DMA,
    ]

    @functools.partial(
        pl.kernel, mesh=mesh, out_type=out_type, scratch_types=scratch,
        compiler_params=pltpu.CompilerParams(needs_layout_passes=False,
                                             use_tc_tiling_on_sc=False))
    def k(hscg, hsgc, stb, scg, dcg, sgc, dgc,
          aggg, deng, aggc, denc,
          rows_v, ssrc_v, sdst_v, exb_v, denb_v, src_v, dst_v,
          agg_s, den_s, sem):
        cid = lax.axis_index("c")
        sid = lax.axis_index("s")

        lane = lax.iota(jnp.int32, 16)
        den_off = jnp.where(lane < 4, lane * K, 0)
        den_msk = jnp.where(lane < 4, 1.0, 0.0).astype(_f32)

        def run(hs, roff, srcE, dstE, aggo, deno):
            def zrow(i, _):
                rows_v[i // 8, pl.ds((i % 8) * 16, 16)] = jnp.zeros((16,), _f32)
                return 0
            lax.fori_loop(0, K * 8, zrow, 0)

            def zden(i, _):
                denb_v[i, :] = jnp.zeros((16,), _f32)
                return 0
            lax.fori_loop(0, K, zden, 0)

            r0 = sid * RW
            def zcp(j, _):
                pltpu.sync_copy(rows_v, agg_s.at[pl.ds(r0 + j * K, K)])
                pltpu.sync_copy(denb_v, den_s.at[pl.ds(r0 + j * K, K)])
                return 0
            lax.fori_loop(0, RW // K, zcp, 0)
            rem = RW - (RW // K) * K
            if rem:
                pltpu.sync_copy(rows_v.at[pl.ds(0, rem)],
                                agg_s.at[pl.ds(r0 + RW - rem, rem)])
                pltpu.sync_copy(denb_v.at[pl.ds(0, rem)],
                                den_s.at[pl.ds(r0 + RW - rem, rem)])

            @pl.when(sid == NSUB - 1)
            def _():
                pltpu.sync_copy(rows_v.at[pl.ds(0, RREM)],
                                agg_s.at[pl.ds(RW * NSUB, RREM)])
                pltpu.sync_copy(denb_v.at[pl.ds(0, RREM)],
                                den_s.at[pl.ds(RW * NSUB, RREM)])
            plsc.subcore_barrier()

            def chunk(ch, _):
                base = sid * EW + ch * K
                pltpu.sync_copy(srcE.at[pl.ds(base, K)], src_v)
                pltpu.sync_copy(dstE.at[pl.ds(base, K)], dst_v)
                c1 = pltpu.async_copy(hs.at[src_v], rows_v, sem)
                c2 = pltpu.async_copy(stb.at[src_v], ssrc_v, sem)
                c3 = pltpu.async_copy(stb.at[dst_v], sdst_v, sem)
                c1.wait()
                c2.wait()
                c3.wait()

                def lgrp(g, _):
                    e0 = g * 16
                    ids = jnp.full((16,), e0, jnp.int32) + lane
                    for h in range(H):
                        av = plsc.load_gather(
                            ssrc_v, [ids, jnp.full((16,), roff + h, jnp.int32)])
                        bv = plsc.load_gather(
                            sdst_v, [ids, jnp.full((16,), roff + 4 + h, jnp.int32)])
                        lv = av + bv
                        lv = jnp.where(lv >= 0.0, lv, 0.2 * lv)
                        exb_v[pl.ds(h * K + e0, 16)] = jnp.exp(lv)
                    return 0
                lax.fori_loop(0, K // 16, lgrp, 0)

                def escale(e, _):
                    esp = jnp.full((16,), e, jnp.int32)
                    dv = plsc.load_gather(exb_v, [esp + den_off]) * den_msk
                    denb_v[e, :] = dv
                    for h in range(H):
                        sp = plsc.load_gather(exb_v, [esp + h * K])
                        for q in range(2):
                            o = h * DH + q * 16
                            rows_v[e, pl.ds(o, 16)] = rows_v[e, pl.ds(o, 16)] * sp
                    return 0
                lax.fori_loop(0, K, escale, 0)

                pltpu.sync_copy(rows_v, agg_s.at[dst_v], add=True)
                pltpu.sync_copy(denb_v, den_s.at[dst_v], add=True)
                return 0
            lax.fori_loop(0, NCH, chunk, 0)
            plsc.subcore_barrier()

            pltpu.sync_copy(agg_s.at[pl.ds(r0, RW)], aggo.at[pl.ds(r0, RW)])
            pltpu.sync_copy(den_s.at[pl.ds(r0, RW)], deno.at[pl.ds(r0, RW)])

            @pl.when(sid == NSUB - 1)
            def _():
                pltpu.sync_copy(agg_s.at[pl.ds(RW * NSUB, RREM)],
                                aggo.at[pl.ds(RW * NSUB, RREM)])
                pltpu.sync_copy(den_s.at[pl.ds(RW * NSUB, RREM)],
                                deno.at[pl.ds(RW * NSUB, RREM)])

        @pl.when(cid == 0)
        def _():
            run(hscg, 0, scg, dcg, aggg, deng)

        @pl.when(cid == 1)
        def _():
            run(hsgc, 8, sgc, dgc, aggc, denc)

    return k(hs_cg, hs_gc, stab, src_cg, dst_cg, src_gc, dst_gc)


def kernel(x_chemical, x_gene, edge_index_cg, edge_index_gc,
           Wsrc, bsrc, Wdst, bdst, attn, Wout, bout):
    xc, xg = x_chemical, x_gene
    src_cg, dst_cg = edge_index_cg[0], edge_index_cg[1]
    src_gc, dst_gc = edge_index_gc[0], edge_index_gc[1]

    eye4 = jnp.eye(H, dtype=_f32)
    Ex = jnp.concatenate(
        [jnp.repeat(eye4, DH, axis=1), jnp.zeros((12, C), _f32)], axis=0)
    z4 = jnp.zeros((C, H), _f32)

    for l in range(L):
        A0 = (attn[l, 0][:, :, None] * eye4[:, None, :]).reshape(C, H)
        A1 = (attn[l, 1][:, :, None] * eye4[:, None, :]).reshape(C, H)
        Wts0, bts0 = Wsrc[l, 0] @ A0, bsrc[l, 0] @ A0
        Wtd0, btd0 = Wdst[l, 0] @ A0, bdst[l, 0] @ A0
        Wts1, bts1 = Wsrc[l, 1] @ A1, bsrc[l, 1] @ A1
        Wtd1, btd1 = Wdst[l, 1] @ A1, bdst[l, 1] @ A1
        Mc = jnp.concatenate([Wts0, z4, z4, Wtd1], axis=1)
        Mg = jnp.concatenate([z4, Wtd0, Wts1, z4], axis=1)
        b16 = jnp.concatenate([bts0, btd0, bts1, btd1])[None]

        hs_cg, hs_gc, stab = _proj_call(
            xc, xg, Wsrc[l, 0], bsrc[l, 0][None], Wsrc[l, 1], bsrc[l, 1][None],
            Mc, Mg, b16)

        aggg, deng, aggc, denc = _sc_edge(
            hs_cg, hs_gc, stab, src_cg, dst_cg, src_gc, dst_gc)

        xg, xc = _out_call(aggg, deng, aggc, denc,
                           Wout[l, 1], bout[l, 1][None],
                           Wout[l, 0], bout[l, 0][None], xg, xc, Ex)

    return jnp.concatenate([xc, xg], axis=0)
